# baseline (speedup 1.0000x reference)
import functools

import jax
import jax.numpy as jnp
from jax.experimental import pallas as pl
from jax.experimental.pallas import tpu as pltpu

_DN = (((1,), (1,)), ((), ()))

_NBUF = 16
_BR = 512


def _proj_kernel(x_ref, w_ref, b_ref, o_hbm, *scratch):
    bufs = scratch[:_NBUF]
    sems = scratch[_NBUF:]
    i = pl.program_id(0)
    nsteps = pl.num_programs(0)
    w = w_ref[...].astype(jnp.bfloat16)
    bias = b_ref[...]
    for s in range(_NBUF):
        row0 = (i * _NBUF + s) * _BR

        @pl.when(i > 0)
        def _wait_prev():
            pltpu.make_async_copy(
                bufs[s], o_hbm.at[pl.ds(row0 - _NBUF * _BR, _BR)], sems[s]
            ).wait()

        x = x_ref[pl.ds(row0, _BR), :].astype(jnp.bfloat16)
        acc = jax.lax.dot_general(x, w, _DN,
                                  preferred_element_type=jnp.float32)
        bufs[s][...] = acc + bias
        pltpu.make_async_copy(
            bufs[s], o_hbm.at[pl.ds(row0, _BR)], sems[s]
        ).start(priority=s % 2)

    @pl.when(i == nsteps - 1)
    def _drain():
        for s in range(_NBUF):
            row0 = (i * _NBUF + s) * _BR
            pltpu.make_async_copy(
                bufs[s], o_hbm.at[pl.ds(row0, _BR)], sems[s]
            ).wait()


@functools.partial(jax.jit, static_argnames=())
def kernel(latents, W, b):
    B, S, D = latents.shape
    E = W.shape[0]
    R = B * S
    x = latents.reshape(R, D)
    b2 = b.reshape(1, E)

    grid = (R // (_BR * _NBUF),)
    out = pl.pallas_call(
        _proj_kernel,
        grid=grid,
        in_specs=[
            pl.BlockSpec((R, D), lambda i: (0, 0)),
            pl.BlockSpec((E, D), lambda i: (0, 0)),
            pl.BlockSpec((1, E), lambda i: (0, 0)),
        ],
        out_specs=pl.BlockSpec(memory_space=pltpu.MemorySpace.HBM),
        out_shape=jax.ShapeDtypeStruct((R, E), jnp.float32),
        scratch_shapes=(
            [pltpu.VMEM((_BR, E), jnp.float32) for _ in range(_NBUF)]
            + [pltpu.SemaphoreType.DMA for _ in range(_NBUF)]
        ),
        compiler_params=pltpu.CompilerParams(
            dimension_semantics=("arbitrary",),
        ),
    )(x, W, b2)
    return out.reshape(B, S, E)


# speedup vs baseline: 1.0327x; 1.0327x over previous
import functools

import jax
import jax.numpy as jnp
from jax.experimental import pallas as pl
from jax.experimental.pallas import tpu as pltpu

_DN = (((1,), (1,)), ((), ()))

_NBUF = 16
_BR = 256


def _proj_kernel(x_ref, w_ref, b_ref, o_hbm, *scratch):
    bufs = scratch[:_NBUF]
    sems = scratch[_NBUF:]
    i = pl.program_id(0)
    nsteps = pl.num_programs(0)
    w = w_ref[...].astype(jnp.bfloat16)
    bias = b_ref[...]
    for s in range(_NBUF):
        row0 = (i * _NBUF + s) * _BR

        @pl.when(i > 0)
        def _wait_prev():
            pltpu.make_async_copy(
                bufs[s], o_hbm.at[pl.ds(row0 - _NBUF * _BR, _BR)], sems[s]
            ).wait()

        x = x_ref[pl.ds(row0, _BR), :].astype(jnp.bfloat16)
        acc = jax.lax.dot_general(x, w, _DN,
                                  preferred_element_type=jnp.float32)
        bufs[s][...] = acc + bias
        pltpu.make_async_copy(
            bufs[s], o_hbm.at[pl.ds(row0, _BR)], sems[s]
        ).start(priority=s % 2)

    @pl.when(i == nsteps - 1)
    def _drain():
        for s in range(_NBUF):
            row0 = (i * _NBUF + s) * _BR
            pltpu.make_async_copy(
                bufs[s], o_hbm.at[pl.ds(row0, _BR)], sems[s]
            ).wait()


@functools.partial(jax.jit, static_argnames=())
def kernel(latents, W, b):
    B, S, D = latents.shape
    E = W.shape[0]
    R = B * S
    x = latents.reshape(R, D)
    b2 = b.reshape(1, E)

    grid = (R // (_BR * _NBUF),)
    out = pl.pallas_call(
        _proj_kernel,
        grid=grid,
        in_specs=[
            pl.BlockSpec((R, D), lambda i: (0, 0)),
            pl.BlockSpec((E, D), lambda i: (0, 0)),
            pl.BlockSpec((1, E), lambda i: (0, 0)),
        ],
        out_specs=pl.BlockSpec(memory_space=pltpu.MemorySpace.HBM),
        out_shape=jax.ShapeDtypeStruct((R, E), jnp.float32),
        scratch_shapes=(
            [pltpu.VMEM((_BR, E), jnp.float32) for _ in range(_NBUF)]
            + [pltpu.SemaphoreType.DMA for _ in range(_NBUF)]
        ),
        compiler_params=pltpu.CompilerParams(
            dimension_semantics=("arbitrary",),
        ),
    )(x, W, b2)
    return out.reshape(B, S, E)
